# Initial kernel scaffold; baseline (speedup 1.0000x reference)
import jax
import jax.numpy as jnp
from jax.experimental import pallas as pl


def kernel(inputs, W_shape, W_color):
    raise NotImplementedError("write your pallas kernel here")


# speedup vs baseline: 3.8472x; 3.8472x over previous
import jax
import jax.numpy as jnp
from jax.experimental import pallas as pl
from jax.experimental.pallas import tpu as pltpu

_NUM_CODES = 1024
_HALF = 64
_COLORS = 16
_CC = 0.25

_T = 2048


def _body(z_ref, ws_ref, wc_ref, q_ref, si_ref, ci_ref, loss_ref):
    i = pl.program_id(0)
    z = z_ref[...]
    zs = z[:, :_HALF]
    zc = z[:, _HALF:]

    ws = ws_ref[...]
    wc = wc_ref[...]
    wsn = ws / jnp.maximum(
        jnp.sqrt(jnp.sum(ws * ws, axis=1, keepdims=True)), 1e-12)
    wcn = wc / jnp.maximum(
        jnp.sqrt(jnp.sum(wc * wc, axis=1, keepdims=True)), 1e-12)
    zsn = zs / jnp.maximum(
        jnp.sqrt(jnp.sum(zs * zs, axis=1, keepdims=True)), 1e-12)
    zcn = zc / jnp.maximum(
        jnp.sqrt(jnp.sum(zc * zc, axis=1, keepdims=True)), 1e-12)

    ls = jnp.dot(zsn, wsn.T, preferred_element_type=jnp.float32)
    lc = jnp.dot(zcn, wcn.T, preferred_element_type=jnp.float32)
    si = jnp.argmax(ls, axis=1).astype(jnp.int32)
    ci = jnp.argmax(lc, axis=1).astype(jnp.int32)

    oh_s = (jax.lax.broadcasted_iota(jnp.int32, (_T, _NUM_CODES), 1)
            == si[:, None]).astype(jnp.float32)
    oh_c = (jax.lax.broadcasted_iota(jnp.int32, (_T, _COLORS), 1)
            == ci[:, None]).astype(jnp.float32)
    qs = jnp.dot(oh_s, wsn, preferred_element_type=jnp.float32)
    qc = jnp.dot(oh_c, wcn, preferred_element_type=jnp.float32)
    q = jnp.concatenate([qs, qc], axis=1)

    q_ref[...] = q
    si_ref[...] = si.reshape(1, 1, _T)
    ci_ref[...] = ci.reshape(1, 1, _T)

    part = jnp.sum((q - z) ** 2).reshape(1, 1)

    @pl.when(i == 0)
    def _():
        loss_ref[...] = jnp.zeros((1, 1), jnp.float32)

    loss_ref[...] += part


def kernel(inputs, W_shape, W_color):
    b, k, d = inputs.shape
    n = b * k
    grid = n // _T
    flat = inputs.reshape(n, d)

    q, si, ci, loss_sum = pl.pallas_call(
        _body,
        grid=(grid,),
        in_specs=[
            pl.BlockSpec((_T, d), lambda i: (i, 0)),
            pl.BlockSpec((_NUM_CODES, _HALF), lambda i: (0, 0)),
            pl.BlockSpec((_COLORS, _HALF), lambda i: (0, 0)),
        ],
        out_specs=[
            pl.BlockSpec((_T, d), lambda i: (i, 0)),
            pl.BlockSpec((1, 1, _T), lambda i: (i, 0, 0)),
            pl.BlockSpec((1, 1, _T), lambda i: (i, 0, 0)),
            pl.BlockSpec((1, 1), lambda i: (0, 0)),
        ],
        out_shape=[
            jax.ShapeDtypeStruct((n, d), jnp.float32),
            jax.ShapeDtypeStruct((grid, 1, _T), jnp.int32),
            jax.ShapeDtypeStruct((grid, 1, _T), jnp.int32),
            jax.ShapeDtypeStruct((1, 1), jnp.float32),
        ],
    )(flat, W_shape, W_color)

    vq_loss = loss_sum[0, 0] * (_CC / (n * d))
    return (q.reshape(b, k, d), vq_loss,
            si.reshape(b, k), ci.reshape(b, k))


# speedup vs baseline: 3.9277x; 1.0209x over previous
import jax
import jax.numpy as jnp
from jax.experimental import pallas as pl
from jax.experimental.pallas import tpu as pltpu

_NUM_CODES = 1024
_HALF = 64
_COLORS = 16
_CC = 0.25

_T = 4096


def _rownorm(w):
    return w / jnp.maximum(
        jnp.sqrt(jnp.sum(w * w, axis=1, keepdims=True)), 1e-12)


def _prep_body(ws_ref, wc_ref, wsn_ref, wcn_ref, bs_ref, bc_ref):
    wsn = _rownorm(ws_ref[...])
    wcn = _rownorm(wc_ref[...])
    wsn_ref[...] = wsn
    wcn_ref[...] = wcn

    io_s = jax.lax.broadcasted_iota(jnp.int32, (_NUM_CODES, 1), 0)
    hi = (io_s >> 6).astype(jnp.bfloat16)
    lo = (io_s & 63).astype(jnp.bfloat16)
    bs_ref[...] = jnp.concatenate(
        [wsn.astype(jnp.bfloat16), hi, lo,
         jnp.zeros((_NUM_CODES, 62), jnp.bfloat16)], axis=1)

    io_c = jax.lax.broadcasted_iota(jnp.int32, (_COLORS, 1), 0)
    bc_ref[...] = jnp.concatenate(
        [wcn.astype(jnp.bfloat16), io_c.astype(jnp.bfloat16),
         jnp.zeros((_COLORS, 63), jnp.bfloat16)], axis=1)


def _body(z_ref, wsn_ref, wcn_ref, bs_ref, bc_ref,
          q_ref, si_ref, ci_ref, loss_ref):
    i = pl.program_id(0)
    z = z_ref[...]
    zs = z[:, :_HALF]
    zc = z[:, _HALF:]
    rs_n = 1.0 / jnp.maximum(
        jnp.sqrt(jnp.sum(zs * zs, axis=1, keepdims=True)), 1e-12)
    rc_n = 1.0 / jnp.maximum(
        jnp.sqrt(jnp.sum(zc * zc, axis=1, keepdims=True)), 1e-12)
    zsn = zs * rs_n
    zcn = zc * rc_n

    ls = jnp.dot(zsn, wsn_ref[...].T,
                 preferred_element_type=jnp.float32)
    lc = jnp.dot(zcn, wcn_ref[...].T,
                 preferred_element_type=jnp.float32)

    ms = (ls == jnp.max(ls, axis=1, keepdims=True)).astype(jnp.bfloat16)
    mc = (lc == jnp.max(lc, axis=1, keepdims=True)).astype(jnp.bfloat16)

    rs = jnp.dot(ms, bs_ref[...], preferred_element_type=jnp.float32)
    rc = jnp.dot(mc, bc_ref[...], preferred_element_type=jnp.float32)

    q = jnp.concatenate([rs[:, :_HALF], rc[:, :_HALF]], axis=1)
    q_ref[...] = q
    si_ref[...] = (rs[:, _HALF:_HALF + 1] * 64.0
                   + rs[:, _HALF + 1:_HALF + 2]).astype(jnp.int32)
    ci_ref[...] = rc[:, _HALF:_HALF + 1].astype(jnp.int32)

    part = jnp.sum((q - z) ** 2).reshape(1, 1)

    @pl.when(i == 0)
    def _():
        loss_ref[...] = jnp.zeros((1, 1), jnp.float32)

    loss_ref[...] += part


def kernel(inputs, W_shape, W_color):
    b, k, d = inputs.shape
    n = b * k
    grid = n // _T
    flat = inputs.reshape(n, d)

    wsn, wcn, bs, bc = pl.pallas_call(
        _prep_body,
        out_shape=[
            jax.ShapeDtypeStruct((_NUM_CODES, _HALF), jnp.float32),
            jax.ShapeDtypeStruct((_COLORS, _HALF), jnp.float32),
            jax.ShapeDtypeStruct((_NUM_CODES, 128), jnp.bfloat16),
            jax.ShapeDtypeStruct((_COLORS, 128), jnp.bfloat16),
        ],
    )(W_shape, W_color)

    q, si, ci, loss_sum = pl.pallas_call(
        _body,
        grid=(grid,),
        in_specs=[
            pl.BlockSpec((_T, d), lambda i: (i, 0)),
            pl.BlockSpec((_NUM_CODES, _HALF), lambda i: (0, 0)),
            pl.BlockSpec((_COLORS, _HALF), lambda i: (0, 0)),
            pl.BlockSpec((_NUM_CODES, 128), lambda i: (0, 0)),
            pl.BlockSpec((_COLORS, 128), lambda i: (0, 0)),
        ],
        out_specs=[
            pl.BlockSpec((_T, d), lambda i: (i, 0)),
            pl.BlockSpec((_T, 1), lambda i: (i, 0)),
            pl.BlockSpec((_T, 1), lambda i: (i, 0)),
            pl.BlockSpec((1, 1), lambda i: (0, 0)),
        ],
        out_shape=[
            jax.ShapeDtypeStruct((n, d), jnp.float32),
            jax.ShapeDtypeStruct((n, 1), jnp.int32),
            jax.ShapeDtypeStruct((n, 1), jnp.int32),
            jax.ShapeDtypeStruct((1, 1), jnp.float32),
        ],
    )(flat, wsn, wcn, bs, bc)

    vq_loss = loss_sum[0, 0] * (_CC / (n * d))
    return (q.reshape(b, k, d), vq_loss,
            si.reshape(b, k), ci.reshape(b, k))
